# baseline (speedup 1.0000x reference)
import jax
import jax.numpy as jnp
from jax.experimental import pallas as pl

K = 2048


def _rownorm(x):
    p = x[:, 0:128] * x[:, 0:128]
    for c in range(1, 4):
        xc = x[:, 128 * c:128 * (c + 1)]
        p = p + xc * xc
    w = p[:, 0:8]
    for m in range(1, 16):
        w = w + p[:, 8 * m:8 * (m + 1)]
    t1 = w[:, 0:4] + w[:, 4:8]
    t2 = t1[:, 0:2] + t1[:, 2:4]
    t3 = t2[:, 0:1] + t2[:, 1:2]
    return jnp.sqrt(t3)


def _norm_body(se_ref, te_ref, sn_ref, tn_ref):
    xs = se_ref[0]
    xt = te_ref[0]
    sn_ref[...] = jnp.transpose(_rownorm(xs))[None]
    tn_ref[...] = jnp.transpose(_rownorm(xt))[None]


def _norms(src_embedding, tgt_embedding):
    B, N, D = src_embedding.shape
    CH = 2048
    nch = N // CH
    grid = (B, nch)
    sn, tn = pl.pallas_call(
        _norm_body,
        grid=grid,
        in_specs=[
            pl.BlockSpec((1, CH, D), lambda b, c: (b, c, 0)),
            pl.BlockSpec((1, CH, D), lambda b, c: (b, c, 0)),
        ],
        out_specs=[
            pl.BlockSpec((1, 1, CH), lambda b, c: (b * nch + c, 0, 0)),
            pl.BlockSpec((1, 1, CH), lambda b, c: (b * nch + c, 0, 0)),
        ],
        out_shape=[
            jax.ShapeDtypeStruct((B * nch, 1, CH), jnp.float32),
            jax.ShapeDtypeStruct((B * nch, 1, CH), jnp.float32),
        ],
    )(src_embedding, tgt_embedding)
    return sn.reshape(B, N), tn.reshape(B, N)


def kernel(src, tgt, n0, n1, src_embedding, tgt_embedding):
    src_norm, tgt_norm = _norms(src_embedding, tgt_embedding)
    _, src_idx = jax.lax.top_k(src_norm, K)
    _, tgt_idx = jax.lax.top_k(tgt_norm, K)
    sidx = src_idx[:, :, None]
    tidx = tgt_idx[:, :, None]
    take = jnp.take_along_axis
    return (take(src, sidx, axis=1),
            take(tgt, tidx, axis=1),
            take(n0, sidx, axis=1),
            take(n1, tidx, axis=1),
            take(src_embedding, sidx, axis=1),
            take(tgt_embedding, tidx, axis=1))


# speedup vs baseline: 1.7986x; 1.7986x over previous
import jax
import jax.numpy as jnp
from jax.experimental import pallas as pl

K = 2048


def _rownorm(x):
    p = x[:, 0:128] * x[:, 0:128]
    for c in range(1, 4):
        xc = x[:, 128 * c:128 * (c + 1)]
        p = p + xc * xc
    pt = jnp.transpose(p)
    w = pt[0:8]
    for m in range(1, 16):
        w = w + pt[8 * m:8 * (m + 1)]
    t1 = w[0:4] + w[4:8]
    t2 = t1[0:2] + t1[2:4]
    t3 = t2[0:1] + t2[1:2]
    return jnp.sqrt(t3)


def _norm_body(se_ref, te_ref, sn_ref, tn_ref):
    xs = se_ref[0]
    xt = te_ref[0]
    sn_ref[...] = _rownorm(xs)[None]
    tn_ref[...] = _rownorm(xt)[None]


def _norms(src_embedding, tgt_embedding):
    B, N, D = src_embedding.shape
    CH = 2048
    nch = N // CH
    grid = (B, nch)
    sn, tn = pl.pallas_call(
        _norm_body,
        grid=grid,
        in_specs=[
            pl.BlockSpec((1, CH, D), lambda b, c: (b, c, 0)),
            pl.BlockSpec((1, CH, D), lambda b, c: (b, c, 0)),
        ],
        out_specs=[
            pl.BlockSpec((1, 1, CH), lambda b, c: (b * nch + c, 0, 0)),
            pl.BlockSpec((1, 1, CH), lambda b, c: (b * nch + c, 0, 0)),
        ],
        out_shape=[
            jax.ShapeDtypeStruct((B * nch, 1, CH), jnp.float32),
            jax.ShapeDtypeStruct((B * nch, 1, CH), jnp.float32),
        ],
    )(src_embedding, tgt_embedding)
    return sn.reshape(B, N), tn.reshape(B, N)


def kernel(src, tgt, n0, n1, src_embedding, tgt_embedding):
    src_norm, tgt_norm = _norms(src_embedding, tgt_embedding)
    _, src_idx = jax.lax.top_k(src_norm, K)
    _, tgt_idx = jax.lax.top_k(tgt_norm, K)
    sidx = src_idx[:, :, None]
    tidx = tgt_idx[:, :, None]
    take = jnp.take_along_axis
    return (take(src, sidx, axis=1),
            take(tgt, tidx, axis=1),
            take(n0, sidx, axis=1),
            take(n1, tidx, axis=1),
            take(src_embedding, sidx, axis=1),
            take(tgt_embedding, tidx, axis=1))
